# Initial kernel scaffold; baseline (speedup 1.0000x reference)
import jax
import jax.numpy as jnp
from jax.experimental import pallas as pl


def kernel(class_logits, box_regression, proposals):
    raise NotImplementedError("write your pallas kernel here")


# speedup vs baseline: 20.5157x; 20.5157x over previous
import functools

import jax
import jax.numpy as jnp
from jax import lax
from jax.experimental import pallas as pl

_NUM_CLASSES = 3
_IMG = 800.0
_SCORE_THRESH = 0.05
_NMS_THRESH = 0.5
_DETS = 100
_CLIP = 4.135166556742356

_N = 20000
_NPAD = 20480
_ROWS = _NPAD // 128
_TROWS = 2 * _ROWS

_NEG_INF = float("-inf")


def _nms_body(i, carry, x1a, y1a, x2a, y2a, areas, idx2d, cls_band, col_iota, row_iota):
    sw, acc = carry
    m = jnp.max(sw)
    picked = m > _NEG_INF
    eq = sw == m
    j = jnp.min(jnp.where(eq, idx2d, jnp.int32(2**30)))
    eqj = idx2d == j
    zero = jnp.zeros_like(x1a)
    bx1 = jnp.sum(jnp.where(eqj, x1a, zero))
    by1 = jnp.sum(jnp.where(eqj, y1a, zero))
    bx2 = jnp.sum(jnp.where(eqj, x2a, zero))
    by2 = jnp.sum(jnp.where(eqj, y2a, zero))
    barea = (bx2 - bx1) * (by2 - by1)
    ltx = jnp.maximum(bx1, x1a)
    lty = jnp.maximum(by1, y1a)
    rbx = jnp.minimum(bx2, x2a)
    rby = jnp.minimum(by2, y2a)
    iw = jnp.maximum(rbx - ltx, 0.0)
    ih = jnp.maximum(rby - lty, 0.0)
    inter = iw * ih
    iou = inter / (barea + areas - inter + 1e-9)
    same_cls = cls_band == (j < _ROWS * 128)
    suppress = ((iou > _NMS_THRESH) & same_cls) | eqj
    sw = jnp.where(suppress, _NEG_INF, sw)
    lab = jnp.where(j < _ROWS * 128, 1.0, 2.0)
    val = jnp.where(row_iota == 0, bx1,
          jnp.where(row_iota == 1, by1,
          jnp.where(row_iota == 2, bx2,
          jnp.where(row_iota == 3, by2,
          jnp.where(row_iota == 4, m, lab)))))
    val = jnp.where(picked, val, 0.0)
    acc = jnp.where(col_iota == i, val, acc)
    return sw, acc


def _fused_kernel(logits_ref, reg_ref, prop_ref, out_ref):
    l0 = logits_ref[0]
    l1 = logits_ref[1]
    l2 = logits_ref[2]
    m = jnp.maximum(jnp.maximum(l0, l1), l2)
    e0 = jnp.exp(l0 - m)
    e1 = jnp.exp(l1 - m)
    e2 = jnp.exp(l2 - m)
    denom = e0 + e1 + e2
    scores = [e1 / denom, e2 / denom]

    px1 = prop_ref[0]
    py1 = prop_ref[1]
    px2 = prop_ref[2]
    py2 = prop_ref[3]
    widths = px2 - px1
    heights = py2 - py1
    ctr_x = px1 + 0.5 * widths
    ctr_y = py1 + 0.5 * heights

    x1s, y1s, x2s, y2s, sws = [], [], [], [], []
    for ci, c in enumerate((1, 2)):
        dx = reg_ref[4 * c + 0] * 0.1
        dy = reg_ref[4 * c + 1] * 0.1
        dw = jnp.minimum(reg_ref[4 * c + 2] * 0.2, _CLIP)
        dh = jnp.minimum(reg_ref[4 * c + 3] * 0.2, _CLIP)
        pcx = dx * widths + ctr_x
        pcy = dy * heights + ctr_y
        pw = jnp.exp(dw) * widths
        ph = jnp.exp(dh) * heights
        x1 = jnp.clip(pcx - 0.5 * pw, 0.0, _IMG)
        y1 = jnp.clip(pcy - 0.5 * ph, 0.0, _IMG)
        x2 = jnp.clip(pcx + 0.5 * pw, 0.0, _IMG)
        y2 = jnp.clip(pcy + 0.5 * ph, 0.0, _IMG)
        s = scores[ci]
        valid = (s > _SCORE_THRESH) & ((x2 - x1) >= 0.01) & ((y2 - y1) >= 0.01)
        sws.append(jnp.where(valid, s, _NEG_INF))
        x1s.append(x1)
        y1s.append(y1)
        x2s.append(x2)
        y2s.append(y2)

    x1a = jnp.concatenate(x1s, axis=0)
    y1a = jnp.concatenate(y1s, axis=0)
    x2a = jnp.concatenate(x2s, axis=0)
    y2a = jnp.concatenate(y2s, axis=0)
    sw = jnp.concatenate(sws, axis=0)
    areas = (x2a - x1a) * (y2a - y1a)

    rid = lax.broadcasted_iota(jnp.int32, (_TROWS, 128), 0)
    cid = lax.broadcasted_iota(jnp.int32, (_TROWS, 128), 1)
    idx2d = rid * 128 + cid
    cls_band = rid < _ROWS

    row8 = lax.broadcasted_iota(jnp.int32, (8, 128), 0)
    col8 = lax.broadcasted_iota(jnp.int32, (8, 128), 1)
    acc0 = jnp.zeros((8, 128), jnp.float32)

    body = functools.partial(
        _nms_body, x1a=x1a, y1a=y1a, x2a=x2a, y2a=y2a, areas=areas,
        idx2d=idx2d, cls_band=cls_band, col_iota=col8, row_iota=row8)
    _, acc = lax.fori_loop(0, _DETS, body, (sw, acc0))
    out_ref[...] = acc


def _run(logits_t, reg_t, prop_t, interpret=False):
    return pl.pallas_call(
        _fused_kernel,
        out_shape=jax.ShapeDtypeStruct((8, 128), jnp.float32),
        interpret=interpret,
    )(logits_t, reg_t, prop_t)


def kernel(class_logits, box_regression, proposals):
    pad = _NPAD - _N
    lt = jnp.pad(class_logits, ((0, pad), (0, 0))).T.reshape(_NUM_CLASSES, _ROWS, 128)
    rt = jnp.pad(box_regression, ((0, pad), (0, 0))).T.reshape(4 * _NUM_CLASSES, _ROWS, 128)
    pt = jnp.pad(proposals, ((0, pad), (0, 0))).T.reshape(4, _ROWS, 128)
    out = _run(lt, rt, pt)
    boxes = out[0:4, :_DETS].T
    nm_scores = out[4, :_DETS]
    labels = out[5, :_DETS].astype(jnp.int32)
    return boxes, nm_scores, labels


# speedup vs baseline: 21.4589x; 1.0460x over previous
import functools

import jax
import jax.numpy as jnp
from jax import lax
from jax.experimental import pallas as pl

_NUM_CLASSES = 3
_IMG = 800.0
_SCORE_THRESH = 0.05
_NMS_THRESH = 0.5
_DETS = 100
_CLIP = 4.135166556742356

_N = 20000
_NPAD = 20480
_ROWS = _NPAD // 128
_TROWS = 2 * _ROWS

_NEG_INF = float("-inf")


def _nms_body(i, carry, x1a, y1a, x2a, y2a, areas, idx2d, cls_band, col_iota, row_iota):
    sw, acc = carry
    m = jnp.max(sw)
    picked = m > _NEG_INF
    eq = sw == m
    j = jnp.min(jnp.where(eq, idx2d, jnp.int32(2**30)))
    eqj = idx2d == j
    zero = jnp.zeros_like(x1a)
    bx1 = jnp.sum(jnp.where(eqj, x1a, zero))
    by1 = jnp.sum(jnp.where(eqj, y1a, zero))
    bx2 = jnp.sum(jnp.where(eqj, x2a, zero))
    by2 = jnp.sum(jnp.where(eqj, y2a, zero))
    barea = (bx2 - bx1) * (by2 - by1)
    ltx = jnp.maximum(bx1, x1a)
    lty = jnp.maximum(by1, y1a)
    rbx = jnp.minimum(bx2, x2a)
    rby = jnp.minimum(by2, y2a)
    iw = jnp.maximum(rbx - ltx, 0.0)
    ih = jnp.maximum(rby - lty, 0.0)
    inter = iw * ih
    iou = inter / (barea + areas - inter + 1e-9)
    same_cls = cls_band == (j < _ROWS * 128)
    suppress = ((iou > _NMS_THRESH) & same_cls) | eqj
    sw = jnp.where(suppress, _NEG_INF, sw)
    lab = jnp.where(j < _ROWS * 128, 1.0, 2.0)
    val = jnp.where(row_iota == 0, bx1,
          jnp.where(row_iota == 1, by1,
          jnp.where(row_iota == 2, bx2,
          jnp.where(row_iota == 3, by2,
          jnp.where(row_iota == 4, m, lab)))))
    val = jnp.where(picked, val, 0.0)
    acc = jnp.where(col_iota == i, val, acc)
    return sw, acc


def _fused_kernel(logits_ref, reg_ref, prop_ref, out_ref):
    l0 = logits_ref[0]
    l1 = logits_ref[1]
    l2 = logits_ref[2]
    m = jnp.maximum(jnp.maximum(l0, l1), l2)
    e0 = jnp.exp(l0 - m)
    e1 = jnp.exp(l1 - m)
    e2 = jnp.exp(l2 - m)
    denom = e0 + e1 + e2
    scores = [e1 / denom, e2 / denom]

    px1 = prop_ref[0]
    py1 = prop_ref[1]
    px2 = prop_ref[2]
    py2 = prop_ref[3]
    widths = px2 - px1
    heights = py2 - py1
    ctr_x = px1 + 0.5 * widths
    ctr_y = py1 + 0.5 * heights

    x1s, y1s, x2s, y2s, sws = [], [], [], [], []
    for ci, c in enumerate((1, 2)):
        dx = reg_ref[4 * c + 0] * 0.1
        dy = reg_ref[4 * c + 1] * 0.1
        dw = jnp.minimum(reg_ref[4 * c + 2] * 0.2, _CLIP)
        dh = jnp.minimum(reg_ref[4 * c + 3] * 0.2, _CLIP)
        pcx = dx * widths + ctr_x
        pcy = dy * heights + ctr_y
        pw = jnp.exp(dw) * widths
        ph = jnp.exp(dh) * heights
        x1 = jnp.clip(pcx - 0.5 * pw, 0.0, _IMG)
        y1 = jnp.clip(pcy - 0.5 * ph, 0.0, _IMG)
        x2 = jnp.clip(pcx + 0.5 * pw, 0.0, _IMG)
        y2 = jnp.clip(pcy + 0.5 * ph, 0.0, _IMG)
        s = scores[ci]
        valid = (s > _SCORE_THRESH) & ((x2 - x1) >= 0.01) & ((y2 - y1) >= 0.01)
        sws.append(jnp.where(valid, s, _NEG_INF))
        x1s.append(x1)
        y1s.append(y1)
        x2s.append(x2)
        y2s.append(y2)

    x1a = jnp.concatenate(x1s, axis=0)
    y1a = jnp.concatenate(y1s, axis=0)
    x2a = jnp.concatenate(x2s, axis=0)
    y2a = jnp.concatenate(y2s, axis=0)
    sw = jnp.concatenate(sws, axis=0)
    areas = (x2a - x1a) * (y2a - y1a)

    rid = lax.broadcasted_iota(jnp.int32, (_TROWS, 128), 0)
    cid = lax.broadcasted_iota(jnp.int32, (_TROWS, 128), 1)
    idx2d = rid * 128 + cid
    cls_band = rid < _ROWS

    row8 = lax.broadcasted_iota(jnp.int32, (8, 128), 0)
    col8 = lax.broadcasted_iota(jnp.int32, (8, 128), 1)
    acc0 = jnp.zeros((8, 128), jnp.float32)

    body = functools.partial(
        _nms_body, x1a=x1a, y1a=y1a, x2a=x2a, y2a=y2a, areas=areas,
        idx2d=idx2d, cls_band=cls_band, col_iota=col8, row_iota=row8)
    _, acc = lax.fori_loop(0, _DETS, body, (sw, acc0))
    out_ref[...] = acc


def _run(logits_t, reg_t, prop_t, interpret=False):
    return pl.pallas_call(
        _fused_kernel,
        out_shape=jax.ShapeDtypeStruct((8, 128), jnp.float32),
        interpret=interpret,
    )(logits_t, reg_t, prop_t)


def kernel(class_logits, box_regression, proposals):
    return kernel_sc(class_logits, box_regression, proposals)


def kernel_tc(class_logits, box_regression, proposals):
    pad = _NPAD - _N
    lt = jnp.pad(class_logits, ((0, pad), (0, 0))).T.reshape(_NUM_CLASSES, _ROWS, 128)
    rt = jnp.pad(box_regression, ((0, pad), (0, 0))).T.reshape(4 * _NUM_CLASSES, _ROWS, 128)
    pt = jnp.pad(proposals, ((0, pad), (0, 0))).T.reshape(4, _ROWS, 128)
    out = _run(lt, rt, pt)
    boxes = out[0:4, :_DETS].T
    nm_scores = out[4, :_DETS]
    labels = out[5, :_DETS].astype(jnp.int32)
    return boxes, nm_scores, labels


from jax.experimental.pallas import tpu as pltpu
from jax.experimental.pallas import tpu_sc as plsc

_CAND = 40960
_CHUNK = 2560
_NTILES = 16
_L1 = _CAND // 16
_L2 = _L1 // 16
_L3 = _L2 // 16
_NPLANES = 19


def _iota16():
    return lax.broadcasted_iota(jnp.int32, (16,), 0)


def _sc_kernel(stacked_hbm, out_hbm, inb, swb, browb, t1, t2, t3,
               accall, boxrow, outb, swp, browp):
    t = lax.axis_index("s")
    iota = _iota16()
    neg = jnp.full((16,), _NEG_INF, jnp.float32)
    zv = jnp.zeros((16,), jnp.float32)

    c = 1 + t // 8
    n0 = (t % 8) * _CHUNK
    planes = [0, 1, 2, -1, -1, -1, -1, 15, 16, 17, 18]
    for k in range(11):
        p = planes[k]
        if p < 0:
            off = (3 + 4 * c + (k - 3)) * _NPAD + n0
        else:
            off = p * _NPAD + n0
        pltpu.sync_copy(stacked_hbm.at[pl.ds(off, _CHUNK)],
                        inb.at[pl.ds(k * _CHUNK, _CHUNK)])

    def decode_body(i, _):
        col = i * 16 + iota
        l0 = plsc.load_gather(inb, [0 * _CHUNK + col])
        l1 = plsc.load_gather(inb, [1 * _CHUNK + col])
        l2 = plsc.load_gather(inb, [2 * _CHUNK + col])
        dx = plsc.load_gather(inb, [3 * _CHUNK + col]) * 0.1
        dy = plsc.load_gather(inb, [4 * _CHUNK + col]) * 0.1
        dw = jnp.minimum(plsc.load_gather(inb, [5 * _CHUNK + col]) * 0.2, _CLIP)
        dh = jnp.minimum(plsc.load_gather(inb, [6 * _CHUNK + col]) * 0.2, _CLIP)
        px1 = plsc.load_gather(inb, [7 * _CHUNK + col])
        py1 = plsc.load_gather(inb, [8 * _CHUNK + col])
        px2 = plsc.load_gather(inb, [9 * _CHUNK + col])
        py2 = plsc.load_gather(inb, [10 * _CHUNK + col])
        w = px2 - px1
        h = py2 - py1
        cx = px1 + 0.5 * w
        cy = py1 + 0.5 * h
        pcx = dx * w + cx
        pcy = dy * h + cy
        pw = jnp.exp(dw) * w
        ph = jnp.exp(dh) * h
        x1 = jnp.clip(pcx - 0.5 * pw, 0.0, _IMG)
        y1 = jnp.clip(pcy - 0.5 * ph, 0.0, _IMG)
        x2 = jnp.clip(pcx + 0.5 * pw, 0.0, _IMG)
        y2 = jnp.clip(pcy + 0.5 * ph, 0.0, _IMG)
        mx = jnp.maximum(jnp.maximum(l0, l1), l2)
        e0 = jnp.exp(l0 - mx)
        e1 = jnp.exp(l1 - mx)
        e2 = jnp.exp(l2 - mx)
        den = e0 + e1 + e2
        s = jnp.where(c == 1, e1, e2) / den
        valid = (s > _SCORE_THRESH) & ((x2 - x1) >= 0.01) & ((y2 - y1) >= 0.01)
        sw = jnp.where(valid, s, neg)
        plsc.store_scatter(swb, [col], sw)
        r8 = col * 8
        plsc.store_scatter(browb, [r8 + 0], x1)
        plsc.store_scatter(browb, [r8 + 1], y1)
        plsc.store_scatter(browb, [r8 + 2], x2)
        plsc.store_scatter(browb, [r8 + 3], y2)
        return 0

    lax.fori_loop(0, _CHUNK // 16, decode_body, 0)

    base = t * _CHUNK
    pltpu.sync_copy(swb, swp.at[pl.ds(base, _CHUNK)])
    pltpu.sync_copy(browb, browp.at[pl.ds(base * 8, _CHUNK * 8)])
    plsc.subcore_barrier()

    @pl.when(t == 0)
    def _nms():
        pltpu.sync_copy(swp, inb)
        for k in range(768 // 16):
            outb[pl.ds(k * 16, 16)] = zv
        for k in range(672 // 16):
            accall[pl.ds(k * 16, 16)] = zv

        lane0 = iota == 0

        def _put(ref, idx, val):
            plsc.store_scatter(ref, [jnp.broadcast_to(idx, (16,))],
                               jnp.broadcast_to(val, (16,)), mask=lane0)

        def l1_body(i, _):
            col = i * 16 + iota
            acc = plsc.load_gather(inb, [col * 16])
            for mi in range(1, 16):
                acc = jnp.maximum(acc, plsc.load_gather(inb, [col * 16 + mi]))
            plsc.store_scatter(t1, [col], acc)
            return 0

        lax.fori_loop(0, _L1 // 16, l1_body, 0)

        def l2_body(i, _):
            col = i * 16 + iota
            acc = plsc.load_gather(t1, [col * 16])
            for mi in range(1, 16):
                acc = jnp.maximum(acc, plsc.load_gather(t1, [col * 16 + mi]))
            plsc.store_scatter(t2, [col], acc)
            return 0

        lax.fori_loop(0, _L2 // 16, l2_body, 0)

        t3v = neg
        for p in range(_L3):
            t3v = jnp.where(iota == p, jnp.max(plsc.load_gather(t2, [p * 16 + iota])), t3v)
        t3[...] = t3v

        m0 = jnp.max(t3[...])

        def pop_cond(carry):
            m, nacc = carry
            return (m > _NEG_INF) & (nacc < _DETS)

        def pop_body(carry):
            m, nacc = carry
            p = jnp.max(plsc.all_reduce_ffs(t3[...] == m))
            v2 = plsc.load_gather(t2, [p * 16 + iota])
            h = p * 16 + jnp.max(plsc.all_reduce_ffs(v2 == m))
            v1 = plsc.load_gather(t1, [h * 16 + iota])
            g = h * 16 + jnp.max(plsc.all_reduce_ffs(v1 == m))
            v0 = plsc.load_gather(inb, [g * 16 + iota])
            j = g * 16 + jnp.max(plsc.all_reduce_ffs(v0 == m))
            pltpu.sync_copy(browp.at[pl.ds(j * 8, 16)], boxrow)
            brow = boxrow[...]
            bx1 = jnp.max(jnp.where(iota == 0, brow, neg))
            by1 = jnp.max(jnp.where(iota == 1, brow, neg))
            bx2 = jnp.max(jnp.where(iota == 2, brow, neg))
            by2 = jnp.max(jnp.where(iota == 3, brow, neg))
            barea = (bx2 - bx1) * (by2 - by1)
            clsj = jnp.where(j < _NPAD, 1.0, 2.0)
            rej = jnp.bool_(False)
            for k in range(_DETS // 16 + 1):
                sl = pl.ds(k * 16, 16)
                a1v = accall[sl]
                b1v = accall[pl.ds(112 + k * 16, 16)]
                a2v = accall[pl.ds(224 + k * 16, 16)]
                b2v = accall[pl.ds(336 + k * 16, 16)]
                aav = accall[pl.ds(448 + k * 16, 16)]
                aclv = accall[pl.ds(560 + k * 16, 16)]
                ltx = jnp.maximum(a1v, bx1)
                lty = jnp.maximum(b1v, by1)
                rbx = jnp.minimum(a2v, bx2)
                rby = jnp.minimum(b2v, by2)
                iw = jnp.maximum(rbx - ltx, 0.0)
                ih = jnp.maximum(rby - lty, 0.0)
                inter = iw * ih
                iou = inter / (aav + barea - inter + 1e-9)
                bad = (iou > _NMS_THRESH) & (aclv == clsj)
                rej = rej | jnp.any(bad)
            accept = jnp.logical_not(rej)

            @pl.when(accept)
            def _store():
                fmask = iota < 6
                vals = jnp.where(iota == 0, bx1,
                       jnp.where(iota == 1, by1,
                       jnp.where(iota == 2, bx2,
                       jnp.where(iota == 3, by2,
                       jnp.where(iota == 4, barea, clsj)))))
                ovals = jnp.where(iota == 4, m, vals)
                plsc.store_scatter(accall, [nacc + 112 * iota], vals, mask=fmask)
                plsc.store_scatter(outb, [nacc + 128 * iota], ovals, mask=fmask)

            _put(inb, j, _NEG_INF)
            _put(t1, g, jnp.max(plsc.load_gather(inb, [g * 16 + iota])))
            _put(t2, h, jnp.max(plsc.load_gather(t1, [h * 16 + iota])))
            _put(t3, p, jnp.max(plsc.load_gather(t2, [p * 16 + iota])))
            m2 = jnp.max(t3[...])
            return m2, nacc + jnp.where(accept, 1, 0)

        lax.while_loop(pop_cond, pop_body, (m0, jnp.int32(0)))
        pltpu.sync_copy(outb, out_hbm)


def _make_sc_call():
    mesh = plsc.VectorSubcoreMesh(core_axis_name="c", subcore_axis_name="s",
                                  num_cores=1)
    f32 = jnp.float32
    return pl.kernel(
        _sc_kernel,
        out_type=jax.ShapeDtypeStruct((768,), f32),
        mesh=mesh,
        compiler_params=pltpu.CompilerParams(needs_layout_passes=False),
        scratch_types=[
            pltpu.VMEM((_CAND,), f32),
            pltpu.VMEM((_CHUNK,), f32),
            pltpu.VMEM((_CHUNK * 8,), f32),
            pltpu.VMEM((_L1,), f32),
            pltpu.VMEM((_L2,), f32),
            pltpu.VMEM((16,), f32),
            pltpu.VMEM((672,), f32),
            pltpu.VMEM((16,), f32),
            pltpu.VMEM((768,), f32),
            pltpu.VMEM_SHARED((_CAND,), f32),
            pltpu.VMEM_SHARED((_CAND * 8 + 8,), f32),
        ],
    )


def kernel_sc(class_logits, box_regression, proposals):
    pad = _NPAD - _N
    lt = jnp.pad(class_logits, ((0, pad), (0, 0))).T
    rt = jnp.pad(box_regression, ((0, pad), (0, 0))).T
    pt = jnp.pad(proposals, ((0, pad), (0, 0))).T
    stacked = jnp.concatenate([lt, rt, pt], axis=0).reshape(-1)
    out = _make_sc_call()(stacked)
    boxes = out[:512].reshape(4, 128)[:, :_DETS].T
    nm_scores = out[512:640][:_DETS]
    labels = out[640:768][:_DETS].astype(jnp.int32)
    return boxes, nm_scores, labels
